# initial kernel scaffold (unmeasured)
import functools

import jax
import jax.numpy as jnp
from jax import lax
from jax.experimental import pallas as pl
from jax.experimental.pallas import tpu as pltpu

P = 8
M = 1024
K = 8192
N = 4096
NBLK = N // P
KT = 1024
NK = K // KT

SERIALIZE = True


def kernel(x, w_mat):
    def body(x_ref, w_ref, out_ref, y_buf, w_buf, w_sems, send_sems,
             recv_sems, out_sem):
        my = lax.axis_index("i")

        barrier_sem = pltpu.get_barrier_semaphore()
        for d in range(P):
            pl.semaphore_signal(
                barrier_sem, inc=1,
                device_id=(d,), device_id_type=pl.DeviceIdType.MESH,
            )
        pl.semaphore_wait(barrier_sem, P)

        def w_dma(t):
            s, k = divmod(t, NK)
            j = (my + s) % P
            return pltpu.make_async_copy(
                w_ref.at[pl.ds(k * KT, KT), pl.ds(j * NBLK, NBLK)],
                w_buf.at[t % 2],
                w_sems.at[t % 2],
            )

        dmas = {0: w_dma(0)}
        dmas[0].start()

        rdmas = []
        out_cp = None
        for s in range(P):
            j = (my + s) % P
            acc = None
            for k in range(NK):
                t = s * NK + k
                if t + 1 < P * NK:
                    dmas[t + 1] = w_dma(t + 1)
                    dmas[t + 1].start()
                dmas[t].wait()
                part = jnp.dot(
                    x_ref[:, k * KT:(k + 1) * KT],
                    w_buf[t % 2],
                    preferred_element_type=jnp.float32,
                )
                acc = part if acc is None else acc + part
            y = acc * (1.0 / (1.0 + jnp.exp(-acc)))
            y_buf[s, :, :] = y

            if s == 0:
                out_cp = pltpu.make_async_copy(
                    y_buf.at[0], out_ref.at[pl.ds(my * M, M), :], out_sem,
                )
                out_cp.start()
            else:
                rdma = pltpu.make_async_remote_copy(
                    src_ref=y_buf.at[s],
                    dst_ref=out_ref.at[pl.ds(my * M, M), :],
                    send_sem=send_sems.at[s],
                    recv_sem=recv_sems.at[s],
                    device_id=(j,),
                    device_id_type=pl.DeviceIdType.MESH,
                )
                rdma.start()
                if SERIALIZE:
                    rdma.wait()
                else:
                    rdmas.append(rdma)

        if not SERIALIZE:
            for rdma in rdmas:
                rdma.wait_send()
            for s in range(1, P):
                src = (my + P - s) % P
                recv = pltpu.make_async_remote_copy(
                    src_ref=y_buf.at[s],
                    dst_ref=out_ref.at[pl.ds(src * M, M), :],
                    send_sem=send_sems.at[s],
                    recv_sem=recv_sems.at[s],
                    device_id=(my,),
                    device_id_type=pl.DeviceIdType.MESH,
                )
                recv.wait_recv()

        out_cp.wait()

        @functools.partial(
            pl.run_scoped, sem2=pltpu.SemaphoreType.REGULAR)
        def _(sem2):
            for d in range(P):
                pl.semaphore_signal(
                    sem2, inc=1,
                    device_id=(d,), device_id_type=pl.DeviceIdType.MESH,
                )
            pl.semaphore_wait(sem2, P)

    return pl.pallas_call(
        body,
        out_shape=jax.ShapeDtypeStruct((P * M, NBLK), jnp.float32),
        in_specs=[
            pl.BlockSpec(memory_space=pltpu.VMEM),
            pl.BlockSpec(memory_space=pltpu.ANY),
        ],
        out_specs=pl.BlockSpec(memory_space=pltpu.ANY),
        scratch_shapes=[
            pltpu.VMEM((P, M, NBLK), jnp.float32),
            pltpu.VMEM((2, KT, NBLK), jnp.float32),
            pltpu.SemaphoreType.DMA((2,)),
            pltpu.SemaphoreType.DMA((P,)),
            pltpu.SemaphoreType.DMA((P,)),
            pltpu.SemaphoreType.DMA,
        ],
        compiler_params=pltpu.CompilerParams(
            collective_id=0,
            vmem_limit_bytes=60 * 1024 * 1024,
        ),
    )(x, w_mat)


# baseline (device time: 307969 ns/iter reference)
import functools

import jax
import jax.numpy as jnp
from jax import lax
from jax.experimental import pallas as pl
from jax.experimental.pallas import tpu as pltpu

P = 8
M = 1024
K = 8192
N = 4096
NBLK = N // P
KT = 1024
NK = K // KT

SERIALIZE = True


def kernel(x, w_mat):
    def body(x_ref, w_ref, out_ref, y_buf, w_buf, w_sems, send_sems,
             recv_sems, out_sem):
        my = lax.axis_index("i")

        barrier_sem = pltpu.get_barrier_semaphore()
        for d in range(P):
            pl.semaphore_signal(
                barrier_sem, inc=1,
                device_id=(d,), device_id_type=pl.DeviceIdType.MESH,
            )
        pl.semaphore_wait(barrier_sem, P)

        def w_dma(t):
            s, k = divmod(t, NK)
            j = (my + s) % P
            return pltpu.make_async_copy(
                w_ref.at[pl.ds(k * KT, KT), pl.ds(j * NBLK, NBLK)],
                w_buf.at[t % 2],
                w_sems.at[t % 2],
            )

        dmas = {0: w_dma(0)}
        dmas[0].start()

        rdmas = []
        out_cp = None
        for s in range(P):
            j = (my + s) % P
            acc = None
            for k in range(NK):
                t = s * NK + k
                if t + 1 < P * NK:
                    dmas[t + 1] = w_dma(t + 1)
                    dmas[t + 1].start()
                dmas[t].wait()
                part = jnp.dot(
                    x_ref[:, k * KT:(k + 1) * KT],
                    w_buf[t % 2],
                    preferred_element_type=jnp.float32,
                )
                acc = part if acc is None else acc + part
            y = acc * (1.0 / (1.0 + jnp.exp(-acc)))
            y_buf[s, :, :] = y

            if s == 0:
                out_cp = pltpu.make_async_copy(
                    y_buf.at[0], out_ref.at[pl.ds(my * M, M), :], out_sem,
                )
                out_cp.start()
            else:
                rdma = pltpu.make_async_remote_copy(
                    src_ref=y_buf.at[s],
                    dst_ref=out_ref.at[pl.ds(my * M, M), :],
                    send_sem=send_sems.at[s],
                    recv_sem=recv_sems.at[s],
                    device_id=(j,),
                    device_id_type=pl.DeviceIdType.MESH,
                )
                rdma.start()
                if SERIALIZE:
                    rdma.wait()
                else:
                    rdmas.append(rdma)

        if not SERIALIZE:
            for rdma in rdmas:
                rdma.wait_send()
            for s in range(1, P):
                src = (my + P - s) % P
                recv = pltpu.make_async_remote_copy(
                    src_ref=y_buf.at[s],
                    dst_ref=out_ref.at[pl.ds(src * M, M), :],
                    send_sem=send_sems.at[s],
                    recv_sem=recv_sems.at[s],
                    device_id=(my,),
                    device_id_type=pl.DeviceIdType.MESH,
                )
                recv.wait_recv()

        out_cp.wait()

        @functools.partial(
            pl.run_scoped, sem2=pltpu.SemaphoreType.REGULAR)
        def _(sem2):
            for d in range(P):
                pl.semaphore_signal(
                    sem2, inc=1,
                    device_id=(d,), device_id_type=pl.DeviceIdType.MESH,
                )
            pl.semaphore_wait(sem2, P)

    return pl.pallas_call(
        body,
        out_shape=jax.ShapeDtypeStruct((P * M, NBLK), jnp.float32),
        in_specs=[
            pl.BlockSpec(memory_space=pltpu.VMEM),
            pl.BlockSpec(memory_space=pl.ANY),
        ],
        out_specs=pl.BlockSpec(memory_space=pl.ANY),
        scratch_shapes=[
            pltpu.VMEM((P, M, NBLK), jnp.float32),
            pltpu.VMEM((2, KT, NBLK), jnp.float32),
            pltpu.SemaphoreType.DMA((2,)),
            pltpu.SemaphoreType.DMA((P,)),
            pltpu.SemaphoreType.DMA((P,)),
            pltpu.SemaphoreType.DMA,
        ],
        compiler_params=pltpu.CompilerParams(
            collective_id=0,
            vmem_limit_bytes=60 * 1024 * 1024,
        ),
    )(x, w_mat)


# device time: 186920 ns/iter; 1.6476x vs baseline; 1.6476x over previous
import functools

import jax
import jax.numpy as jnp
from jax import lax
from jax.experimental import pallas as pl
from jax.experimental.pallas import tpu as pltpu

P = 8
M = 1024
K = 8192
N = 4096
NBLK = N // P
KT = 1024
NK = K // KT

SERIALIZE = False


def kernel(x, w_mat):
    def body(x_ref, w_ref, out_ref, y_buf, w_buf, w_sems, send_sems,
             recv_sems, out_sem):
        my = lax.axis_index("i")

        barrier_sem = pltpu.get_barrier_semaphore()
        for d in range(P):
            pl.semaphore_signal(
                barrier_sem, inc=1,
                device_id=(d,), device_id_type=pl.DeviceIdType.MESH,
            )
        pl.semaphore_wait(barrier_sem, P)

        def w_dma(t):
            s, k = divmod(t, NK)
            j = (my + s) % P
            return pltpu.make_async_copy(
                w_ref.at[pl.ds(k * KT, KT), pl.ds(j * NBLK, NBLK)],
                w_buf.at[t % 2],
                w_sems.at[t % 2],
            )

        dmas = {0: w_dma(0)}
        dmas[0].start()

        rdmas = []
        out_cp = None
        for s in range(P):
            j = (my + s) % P
            acc = None
            for k in range(NK):
                t = s * NK + k
                if t + 1 < P * NK:
                    dmas[t + 1] = w_dma(t + 1)
                    dmas[t + 1].start()
                dmas[t].wait()
                part = jnp.dot(
                    x_ref[:, k * KT:(k + 1) * KT],
                    w_buf[t % 2],
                    preferred_element_type=jnp.float32,
                )
                acc = part if acc is None else acc + part
            y = acc * (1.0 / (1.0 + jnp.exp(-acc)))
            y_buf[s, :, :] = y

            if s == 0:
                out_cp = pltpu.make_async_copy(
                    y_buf.at[0], out_ref.at[pl.ds(my * M, M), :], out_sem,
                )
                out_cp.start()
            else:
                rdma = pltpu.make_async_remote_copy(
                    src_ref=y_buf.at[s],
                    dst_ref=out_ref.at[pl.ds(my * M, M), :],
                    send_sem=send_sems.at[s],
                    recv_sem=recv_sems.at[s],
                    device_id=(j,),
                    device_id_type=pl.DeviceIdType.MESH,
                )
                rdma.start()
                if SERIALIZE:
                    rdma.wait()
                else:
                    rdmas.append(rdma)

        if not SERIALIZE:
            for rdma in rdmas:
                rdma.wait_send()
            for s in range(1, P):
                src = (my + P - s) % P
                recv = pltpu.make_async_remote_copy(
                    src_ref=y_buf.at[s],
                    dst_ref=out_ref.at[pl.ds(src * M, M), :],
                    send_sem=send_sems.at[s],
                    recv_sem=recv_sems.at[s],
                    device_id=(my,),
                    device_id_type=pl.DeviceIdType.MESH,
                )
                recv.wait_recv()

        out_cp.wait()

        @functools.partial(
            pl.run_scoped, sem2=pltpu.SemaphoreType.REGULAR)
        def _(sem2):
            for d in range(P):
                pl.semaphore_signal(
                    sem2, inc=1,
                    device_id=(d,), device_id_type=pl.DeviceIdType.MESH,
                )
            pl.semaphore_wait(sem2, P)

    return pl.pallas_call(
        body,
        out_shape=jax.ShapeDtypeStruct((P * M, NBLK), jnp.float32),
        in_specs=[
            pl.BlockSpec(memory_space=pltpu.VMEM),
            pl.BlockSpec(memory_space=pl.ANY),
        ],
        out_specs=pl.BlockSpec(memory_space=pl.ANY),
        scratch_shapes=[
            pltpu.VMEM((P, M, NBLK), jnp.float32),
            pltpu.VMEM((2, KT, NBLK), jnp.float32),
            pltpu.SemaphoreType.DMA((2,)),
            pltpu.SemaphoreType.DMA((P,)),
            pltpu.SemaphoreType.DMA((P,)),
            pltpu.SemaphoreType.DMA,
        ],
        compiler_params=pltpu.CompilerParams(
            collective_id=0,
            vmem_limit_bytes=60 * 1024 * 1024,
        ),
    )(x, w_mat)


# device time: 163403 ns/iter; 1.8847x vs baseline; 1.1439x over previous
import functools
import os

import jax
import jax.numpy as jnp
from jax import lax
from jax.experimental import pallas as pl
from jax.experimental.pallas import tpu as pltpu

P = 8
M = 1024
K = 8192
N = 4096
NBLK = N // P
KT = 1024
NK = K // KT

SERIALIZE = False
NO_COMM = bool(os.environ.get("A2A_NO_COMM"))
NO_COMPUTE = bool(os.environ.get("A2A_NO_COMPUTE"))


def kernel(x, w_mat):
    def body(x_ref, w_ref, out_ref, y_buf, w_buf, w_sems, send_sems,
             recv_sems, out_sem):
        my = lax.axis_index("i")

        barrier_sem = pltpu.get_barrier_semaphore()
        for d in range(P):
            pl.semaphore_signal(
                barrier_sem, inc=1,
                device_id=(d,), device_id_type=pl.DeviceIdType.MESH,
            )
        pl.semaphore_wait(barrier_sem, P)

        def w_dma(t):
            s, k = divmod(t, NK)
            j = (my + s) % P
            return pltpu.make_async_copy(
                w_ref.at[pl.ds(k * KT, KT), pl.ds(j * NBLK, NBLK)],
                w_buf.at[t % 2],
                w_sems.at[t % 2],
            )

        if not NO_COMPUTE:
            dmas = {0: w_dma(0)}
            dmas[0].start()

        rdmas = []
        out_cp = None
        for s in range(P):
            j = (my + s) % P
            if not NO_COMPUTE:
                acc = None
                for k in range(NK):
                    t = s * NK + k
                    if t + 1 < P * NK:
                        dmas[t + 1] = w_dma(t + 1)
                        dmas[t + 1].start()
                    dmas[t].wait()
                    part = jnp.dot(
                        x_ref[:, k * KT:(k + 1) * KT],
                        w_buf[t % 2],
                        preferred_element_type=jnp.float32,
                    )
                    acc = part if acc is None else acc + part
                y = acc * (1.0 / (1.0 + jnp.exp(-acc)))
                y_buf[s, :, :] = y
            if NO_COMM and s > 0:
                continue

            if s == 0:
                out_cp = pltpu.make_async_copy(
                    y_buf.at[0], out_ref.at[pl.ds(my * M, M), :], out_sem,
                )
                out_cp.start()
            else:
                rdma = pltpu.make_async_remote_copy(
                    src_ref=y_buf.at[s],
                    dst_ref=out_ref.at[pl.ds(my * M, M), :],
                    send_sem=send_sems.at[s],
                    recv_sem=recv_sems.at[s],
                    device_id=(j,),
                    device_id_type=pl.DeviceIdType.MESH,
                )
                rdma.start()
                if SERIALIZE:
                    rdma.wait()
                else:
                    rdmas.append(rdma)

        if not SERIALIZE and not NO_COMM:
            for rdma in rdmas:
                rdma.wait_send()
            for s in range(1, P):
                src = (my + P - s) % P
                recv = pltpu.make_async_remote_copy(
                    src_ref=y_buf.at[s],
                    dst_ref=out_ref.at[pl.ds(src * M, M), :],
                    send_sem=send_sems.at[s],
                    recv_sem=recv_sems.at[s],
                    device_id=(my,),
                    device_id_type=pl.DeviceIdType.MESH,
                )
                recv.wait_recv()

        out_cp.wait()

        @functools.partial(
            pl.run_scoped, sem2=pltpu.SemaphoreType.REGULAR)
        def _(sem2):
            for d in range(P):
                pl.semaphore_signal(
                    sem2, inc=1,
                    device_id=(d,), device_id_type=pl.DeviceIdType.MESH,
                )
            pl.semaphore_wait(sem2, P)

    return pl.pallas_call(
        body,
        out_shape=jax.ShapeDtypeStruct((P * M, NBLK), jnp.float32),
        in_specs=[
            pl.BlockSpec(memory_space=pltpu.VMEM),
            pl.BlockSpec(memory_space=pl.ANY),
        ],
        out_specs=pl.BlockSpec(memory_space=pl.ANY),
        scratch_shapes=[
            pltpu.VMEM((P, M, NBLK), jnp.float32),
            pltpu.VMEM((2, KT, NBLK), jnp.float32),
            pltpu.SemaphoreType.DMA((2,)),
            pltpu.SemaphoreType.DMA((P,)),
            pltpu.SemaphoreType.DMA((P,)),
            pltpu.SemaphoreType.DMA,
        ],
        compiler_params=pltpu.CompilerParams(
            collective_id=0,
            vmem_limit_bytes=60 * 1024 * 1024,
        ),
    )(x, w_mat)


# device time: 145423 ns/iter; 2.1177x vs baseline; 1.1236x over previous
import functools
import os

import jax
import jax.numpy as jnp
from jax import lax
from jax.experimental import pallas as pl
from jax.experimental.pallas import tpu as pltpu

P = 8
M = 1024
K = 8192
N = 4096
NBLK = N // P
KT = 1024
NK = K // KT

SERIALIZE = False
NO_COMM = bool(os.environ.get("A2A_NO_COMM"))
NO_COMPUTE = bool(os.environ.get("A2A_NO_COMPUTE"))


def kernel(x, w_mat):
    def body(x_ref, w_ref, out_ref, xb, xin, y_send, own_buf, recv_buf,
             stage, w_buf, x_sems, w_sems, send_sems, recv_sems, stage_sems,
             out_sem):
        my = lax.axis_index("i")

        barrier_sem = pltpu.get_barrier_semaphore()
        for d in range(P):
            pl.semaphore_signal(
                barrier_sem, inc=1,
                device_id=(d,), device_id_type=pl.DeviceIdType.MESH,
            )
        pl.semaphore_wait(barrier_sem, P)

        def w_dma(t):
            s, k = divmod(t, NK)
            j = (my + s) % P
            return pltpu.make_async_copy(
                w_ref.at[pl.ds(k * KT, KT), pl.ds(j * NBLK, NBLK)],
                w_buf.at[t % 2],
                w_sems.at[t % 2],
            )

        def x_dma(k):
            return pltpu.make_async_copy(
                x_ref.at[:, pl.ds(k * KT, KT)],
                xin.at[k % 2],
                x_sems.at[k % 2],
            )

        if not NO_COMPUTE:
            w_dmas = {0: w_dma(0)}
            w_dmas[0].start()
            x_dmas = {0: x_dma(0)}
            x_dmas[0].start()

        rdmas = []
        out_cp = None
        for s in range(P):
            j = (my + s) % P
            if not NO_COMPUTE:
                acc = None
                for k in range(NK):
                    t = s * NK + k
                    if t + 1 < P * NK:
                        w_dmas[t + 1] = w_dma(t + 1)
                        w_dmas[t + 1].start()
                    if s == 0:
                        if k + 1 < NK:
                            x_dmas[k + 1] = x_dma(k + 1)
                            x_dmas[k + 1].start()
                        x_dmas[k].wait()
                        xb[:, k * KT:(k + 1) * KT] = xin[k % 2].astype(
                            jnp.bfloat16)
                    w_dmas[t].wait()
                    part = jnp.dot(
                        xb[:, k * KT:(k + 1) * KT],
                        w_buf[t % 2].astype(jnp.bfloat16),
                        preferred_element_type=jnp.float32,
                    )
                    acc = part if acc is None else acc + part
                y = acc * (1.0 / (1.0 + jnp.exp(-acc)))
                if s == 0:
                    own_buf[:, :] = y
                else:
                    y_send[s, :, :] = y.astype(jnp.bfloat16)
            if NO_COMM and s > 0:
                continue

            if s == 0:
                out_cp = pltpu.make_async_copy(
                    own_buf, out_ref.at[pl.ds(my * M, M), :], out_sem,
                )
                out_cp.start()
            else:
                rdma = pltpu.make_async_remote_copy(
                    src_ref=y_send.at[s],
                    dst_ref=recv_buf.at[s],
                    send_sem=send_sems.at[s],
                    recv_sem=recv_sems.at[s],
                    device_id=(j,),
                    device_id_type=pl.DeviceIdType.MESH,
                )
                rdma.start()
                if SERIALIZE:
                    rdma.wait()
                rdmas.append(rdma)

        if not NO_COMM:
            out_dmas = {}
            for s in range(1, P):
                src = (my + P - s) % P
                if not SERIALIZE:
                    recv = pltpu.make_async_remote_copy(
                        src_ref=y_send.at[s],
                        dst_ref=recv_buf.at[s],
                        send_sem=send_sems.at[s],
                        recv_sem=recv_sems.at[s],
                        device_id=(my,),
                        device_id_type=pl.DeviceIdType.MESH,
                    )
                    recv.wait_recv()
                if s - 2 in out_dmas:
                    out_dmas[s - 2].wait()
                stage[s % 2, :, :] = recv_buf[s].astype(jnp.float32)
                out_dmas[s] = pltpu.make_async_copy(
                    stage.at[s % 2],
                    out_ref.at[pl.ds(src * M, M), :],
                    stage_sems.at[s % 2],
                )
                out_dmas[s].start()
            for s in (P - 2, P - 1):
                if s in out_dmas:
                    out_dmas[s].wait()
            for rdma in rdmas:
                rdma.wait_send()

        out_cp.wait()

        @functools.partial(
            pl.run_scoped, sem2=pltpu.SemaphoreType.REGULAR)
        def _(sem2):
            for d in range(P):
                pl.semaphore_signal(
                    sem2, inc=1,
                    device_id=(d,), device_id_type=pl.DeviceIdType.MESH,
                )
            pl.semaphore_wait(sem2, P)

    return pl.pallas_call(
        body,
        out_shape=jax.ShapeDtypeStruct((P * M, NBLK), jnp.float32),
        in_specs=[
            pl.BlockSpec(memory_space=pl.ANY),
            pl.BlockSpec(memory_space=pl.ANY),
        ],
        out_specs=pl.BlockSpec(memory_space=pl.ANY),
        scratch_shapes=[
            pltpu.VMEM((M, K), jnp.bfloat16),
            pltpu.VMEM((2, M, KT), jnp.float32),
            pltpu.VMEM((P, M, NBLK), jnp.bfloat16),
            pltpu.VMEM((M, NBLK), jnp.float32),
            pltpu.VMEM((P, M, NBLK), jnp.bfloat16),
            pltpu.VMEM((2, M, NBLK), jnp.float32),
            pltpu.VMEM((2, KT, NBLK), jnp.float32),
            pltpu.SemaphoreType.DMA((2,)),
            pltpu.SemaphoreType.DMA((2,)),
            pltpu.SemaphoreType.DMA((P,)),
            pltpu.SemaphoreType.DMA((P,)),
            pltpu.SemaphoreType.DMA((2,)),
            pltpu.SemaphoreType.DMA,
        ],
        compiler_params=pltpu.CompilerParams(
            collective_id=0,
            vmem_limit_bytes=60 * 1024 * 1024,
        ),
    )(x, w_mat)


# device time: 143314 ns/iter; 2.1489x vs baseline; 1.0147x over previous
import functools
import os

import jax
import jax.numpy as jnp
from jax import lax
from jax.experimental import pallas as pl
from jax.experimental.pallas import tpu as pltpu

P = 8
M = 1024
K = 8192
N = 4096
NBLK = N // P
KT = 1024
NK = K // KT
NPAIR = P // 2
WB = 2

NO_COMM = bool(os.environ.get("A2A_NO_COMM"))
NO_COMPUTE = bool(os.environ.get("A2A_NO_COMPUTE"))


def kernel(x, w_mat):
    def body(x_ref, w_ref, out_ref, xb, xin, y_send, recv_buf, stage,
             w_buf, wbf, acc, x_sem, w_sems, send_sems, recv_sems,
             stage_sems):
        my = lax.axis_index("i")

        barrier_sem = pltpu.get_barrier_semaphore()
        for d in range(P):
            pl.semaphore_signal(
                barrier_sem, inc=1,
                device_id=(d,), device_id_type=pl.DeviceIdType.MESH,
            )
        pl.semaphore_wait(barrier_sem, P)

        def w_dma(t):
            i, k = divmod(t, NK)
            q = (my // 2 + i) % NPAIR
            return pltpu.make_async_copy(
                w_ref.at[pl.ds(k * KT, KT), pl.ds(q * 2 * NBLK, 2 * NBLK)],
                w_buf.at[t % WB],
                w_sems.at[t % WB],
            )

        def x_dma(k):
            return pltpu.make_async_copy(
                x_ref.at[:, pl.ds(k * KT, KT)], xin, x_sem,
            )

        if not NO_COMPUTE:
            for t0 in range(WB - 1):
                w_dma(t0).start()
            x_dma(0).start()

        def pair_body(i, carry):
            q = (my // 2 + i) % NPAIR
            j0 = 2 * q
            if not NO_COMPUTE:
                for k in range(NK):
                    t = i * NK + k
                    @pl.when(t + WB - 1 < NPAIR * NK)
                    def _():
                        w_dma(t + WB - 1).start()

                    @pl.when(i == 0)
                    def _():
                        x_dma(k).wait()
                        xb[:, k * KT:(k + 1) * KT] = xin[:, :].astype(
                            jnp.bfloat16)
                        if k + 1 < NK:
                            x_dma(k + 1).start()
                    w_dma(t).wait()
                    wbf[:, :] = w_buf[t % WB].astype(jnp.bfloat16)
                    for d in (0, 1):
                        part = jnp.dot(
                            xb[:, k * KT:(k + 1) * KT],
                            wbf[:, d * NBLK:(d + 1) * NBLK],
                            preferred_element_type=jnp.float32,
                        )
                        if k == 0:
                            acc[d, :, :] = part
                        else:
                            acc[d, :, :] = acc[d, :, :] + part
                for d in (0, 1):
                    y = acc[d]
                    y = y * (1.0 / (1.0 + jnp.exp(-y)))
                    y_send[2 * i + d, :, :] = y.astype(jnp.bfloat16)
            if not NO_COMM:
                for d in (0, 1):
                    pltpu.make_async_remote_copy(
                        src_ref=y_send.at[2 * i + d],
                        dst_ref=recv_buf.at[my],
                        send_sem=send_sems.at[2 * i + d],
                        recv_sem=recv_sems.at[my],
                        device_id=(j0 + d,),
                        device_id_type=pl.DeviceIdType.MESH,
                    ).start()
            return carry

        lax.fori_loop(0, NPAIR, pair_body, 0)

        if not NO_COMM:
            out_dmas = {}
            for s in range(P):
                recv = pltpu.make_async_remote_copy(
                    src_ref=y_send.at[0],
                    dst_ref=recv_buf.at[s],
                    send_sem=send_sems.at[0],
                    recv_sem=recv_sems.at[s],
                    device_id=(my,),
                    device_id_type=pl.DeviceIdType.MESH,
                )
                recv.wait_recv()
                if s - 1 in out_dmas:
                    out_dmas[s - 1].wait()
                stage[:, :] = recv_buf[s].astype(jnp.float32)
                out_dmas[s] = pltpu.make_async_copy(
                    stage,
                    out_ref.at[pl.ds(s * M, M), :],
                    stage_sems,
                )
                out_dmas[s].start()
            out_dmas[P - 1].wait()
            for s in range(P):
                pltpu.make_async_remote_copy(
                    src_ref=y_send.at[s],
                    dst_ref=recv_buf.at[s],
                    send_sem=send_sems.at[s],
                    recv_sem=recv_sems.at[s],
                    device_id=(my,),
                    device_id_type=pl.DeviceIdType.MESH,
                ).wait_send()

        @functools.partial(
            pl.run_scoped, sem2=pltpu.SemaphoreType.REGULAR)
        def _(sem2):
            for d in range(P):
                pl.semaphore_signal(
                    sem2, inc=1,
                    device_id=(d,), device_id_type=pl.DeviceIdType.MESH,
                )
            pl.semaphore_wait(sem2, P)

    return pl.pallas_call(
        body,
        out_shape=jax.ShapeDtypeStruct((P * M, NBLK), jnp.float32),
        in_specs=[
            pl.BlockSpec(memory_space=pl.ANY),
            pl.BlockSpec(memory_space=pl.ANY),
        ],
        out_specs=pl.BlockSpec(memory_space=pl.ANY),
        scratch_shapes=[
            pltpu.VMEM((M, K), jnp.bfloat16),
            pltpu.VMEM((M, KT), jnp.float32),
            pltpu.VMEM((P, M, NBLK), jnp.bfloat16),
            pltpu.VMEM((P, M, NBLK), jnp.bfloat16),
            pltpu.VMEM((M, NBLK), jnp.float32),
            pltpu.VMEM((WB, KT, 2 * NBLK), jnp.float32),
            pltpu.VMEM((KT, 2 * NBLK), jnp.bfloat16),
            pltpu.VMEM((2, M, NBLK), jnp.float32),
            pltpu.SemaphoreType.DMA,
            pltpu.SemaphoreType.DMA((WB,)),
            pltpu.SemaphoreType.DMA((P,)),
            pltpu.SemaphoreType.DMA((P,)),
            pltpu.SemaphoreType.DMA,
        ],
        compiler_params=pltpu.CompilerParams(
            collective_id=0,
            vmem_limit_bytes=64 * 1024 * 1024,
        ),
    )(x, w_mat)
